# baseline (speedup 1.0000x reference)
import functools

import jax
import jax.numpy as jnp
from jax import lax
from jax.experimental import pallas as pl
from jax.experimental.pallas import tpu as pltpu
from jax.experimental.pallas import tpu_sc as plsc

VOCAB = 1000000
C0, C1 = 20000, 200000
D = 128
N0, D0 = 20000, 128
N1, D1 = 180000, 32
N2, D2 = 800000, 8

T = 4096 * 50
NS = 2
H = T // NS
NW = 32
TPW = H // NW
L = 16

R0C = 128
R1C = 512
R2C = 1024

HPAD = H + 256

BT = 2048


def _sc_gather(ids, emb0, emb1, emb2):
    mesh = plsc.VectorSubcoreMesh(core_axis_name="c", subcore_axis_name="s")

    @functools.partial(
        pl.kernel,
        out_type=(
            jax.ShapeDtypeStruct((HPAD, D0), jnp.float32),
            jax.ShapeDtypeStruct((HPAD, D1), jnp.float32),
            jax.ShapeDtypeStruct((HPAD, D2), jnp.float32),
        ),
        mesh=mesh,
        compiler_params=pltpu.CompilerParams(
            use_tc_tiling_on_sc=False, needs_layout_passes=False),
        scratch_types=[
            pltpu.VMEM((TPW,), jnp.int32),
            pltpu.VMEM((TPW + R0C,), jnp.int32),
            pltpu.VMEM((TPW + R0C,), jnp.int32),
            pltpu.VMEM((TPW + R1C,), jnp.int32),
            pltpu.VMEM((TPW + R1C,), jnp.int32),
            pltpu.VMEM((TPW + R2C,), jnp.int32),
            pltpu.VMEM((TPW + R2C,), jnp.int32),
            pltpu.VMEM((R0C, D0), jnp.float32),
            pltpu.VMEM((R1C, D1), jnp.float32),
            pltpu.VMEM((R2C, D2), jnp.float32),
            pltpu.VMEM((R0C,), jnp.int32),
            pltpu.VMEM((R1C,), jnp.int32),
            pltpu.VMEM((R2C,), jnp.int32),
            pltpu.SemaphoreType.DMA,
        ],
    )
    def k(ids_hbm, e0_hbm, e1_hbm, e2_hbm, g0s_hbm, g1s_hbm, g2s_hbm,
          ids_v, p0_v, q0_v, p1_v, q1_v, p2_v, q2_v,
          rows0_v, rows1_v, rows2_v, pc0_v, pc1_v, pc2_v, sem):
        wid = lax.axis_index("s") * 2 + lax.axis_index("c")
        base = wid * TPW
        pltpu.sync_copy(ids_hbm.at[pl.ds(base, TPW)], ids_v)

        def idx_body(g, offs):
            o0, o1, o2 = offs
            v = ids_v[pl.ds(g * L, L)]
            pos = base + g * L + lax.iota(jnp.int32, L)
            m0 = (v < C0) & (v != 0)
            m1 = (v >= C0) & (v < C1)
            m2 = v >= C1
            outs = []
            for (m, p_v, q_v, o, sub) in (
                    (m0, p0_v, q0_v, o0, 0),
                    (m1, p1_v, q1_v, o1, C0),
                    (m2, p2_v, q2_v, o2, C1)):
                mi = m.astype(jnp.int32)
                tgt = o + plsc.cumsum(mi) - 1
                plsc.store_scatter(p_v, [tgt], pos, mask=m)
                plsc.store_scatter(q_v, [tgt], v - sub, mask=m)
                outs.append(o + jnp.sum(mi))
            return tuple(outs)

        o0, o1, o2 = lax.fori_loop(0, TPW // L, idx_body, (0, 0, 0))

        for (p_v, q_v, o, rc) in ((p0_v, q0_v, o0, R0C),
                                  (p1_v, q1_v, o1, R1C),
                                  (p2_v, q2_v, o2, R2C)):
            for u in range(rc // L):
                p_v[pl.ds(o + u * L, L)] = jnp.full((L,), H, jnp.int32)
                q_v[pl.ds(o + u * L, L)] = jnp.zeros((L,), jnp.int32)

        def make_round(e_hbm, gs_hbm, p_v, q_v, rows_v, pc_v, rc):
            def rnd(j, c):
                cb = j * rc
                pltpu.async_copy(
                    e_hbm.at[q_v.at[pl.ds(cb, rc)]], rows_v, sem).wait()
                for u in range(rc // L):
                    pc_v[pl.ds(u * L, L)] = p_v[pl.ds(cb + u * L, L)]
                pltpu.async_copy(rows_v, gs_hbm.at[pc_v], sem).wait()
                return c
            return rnd

        lax.fori_loop(0, (o0 + R0C - 1) // R0C,
                      make_round(e0_hbm, g0s_hbm, p0_v, q0_v, rows0_v,
                                 pc0_v, R0C), 0)
        lax.fori_loop(0, (o1 + R1C - 1) // R1C,
                      make_round(e1_hbm, g1s_hbm, p1_v, q1_v, rows1_v,
                                 pc1_v, R1C), 0)
        lax.fori_loop(0, (o2 + R2C - 1) // R2C,
                      make_round(e2_hbm, g2s_hbm, p2_v, q2_v, rows2_v,
                                 pc2_v, R2C), 0)

    return k(ids, emb0, emb1, emb2)


def _tc_combine_body(ids_ref, g0_ref, g1_ref, g2_ref, w1_ref, w2_ref,
                     b1_ref, b2_ref, o_ref):
    ids1 = ids_ref[...]
    idsb = jnp.broadcast_to(ids1, (ids1.shape[0], D))
    ids32 = idsb[:, :D1]
    ids8 = idsb[:, :D2]
    g0 = jnp.where((idsb < C0) & (idsb != 0), g0_ref[...], 0.0)
    g1 = jnp.where((ids32 >= C0) & (ids32 < C1) & (ids32 != C0),
                   g1_ref[...], 0.0)
    g2 = jnp.where(ids8 >= C1, g2_ref[...], 0.0)
    g2 = jnp.where(ids8 != C1, g2, 0.0)
    acc = g0
    acc = acc + jnp.dot(g1, w1_ref[...], preferred_element_type=jnp.float32)
    acc = acc + jnp.dot(g2, w2_ref[...], preferred_element_type=jnp.float32)
    acc = acc + jnp.where((idsb >= C0) & (idsb < C1), b1_ref[...], 0.0)
    acc = acc + jnp.where(idsb >= C1, b2_ref[...], 0.0)
    o_ref[...] = acc


def _tc_combine(ids2d, g0s, g1s, g2s, w1t, w2t, b1, b2):
    return pl.pallas_call(
        _tc_combine_body,
        grid=(H // BT,),
        in_specs=[
            pl.BlockSpec((BT, 1), lambda i: (i, 0)),
            pl.BlockSpec((BT, D0), lambda i: (i, 0)),
            pl.BlockSpec((BT, D1), lambda i: (i, 0)),
            pl.BlockSpec((BT, D2), lambda i: (i, 0)),
            pl.BlockSpec((D1, D), lambda i: (0, 0)),
            pl.BlockSpec((D2, D), lambda i: (0, 0)),
            pl.BlockSpec((1, D), lambda i: (0, 0)),
            pl.BlockSpec((1, D), lambda i: (0, 0)),
        ],
        out_specs=pl.BlockSpec((BT, D), lambda i: (i, 0)),
        out_shape=jax.ShapeDtypeStruct((H, D), jnp.float32),
    )(ids2d, g0s, g1s, g2s, w1t, w2t, b1, b2)


def kernel(input_ids, emb0, emb1, emb2, proj1_w, proj1_b, proj2_w, proj2_b):
    ids = input_ids.reshape(-1).astype(jnp.int32)
    w1t, w2t = proj1_w.T, proj2_w.T
    b1, b2 = proj1_b.reshape(1, D), proj2_b.reshape(1, D)
    halves = []
    for h in range(NS):
        idh = ids[h * H:(h + 1) * H]
        g0s, g1s, g2s = _sc_gather(idh, emb0, emb1, emb2)
        halves.append(_tc_combine(idh.reshape(H, 1), g0s, g1s, g2s,
                                  w1t, w2t, b1, b2))
    out = jnp.concatenate(halves, axis=0)
    return out.reshape(input_ids.shape + (D,))


# speedup vs baseline: 1.1319x; 1.1319x over previous
import functools

import jax
import jax.numpy as jnp
from jax import lax
from jax.experimental import pallas as pl
from jax.experimental.pallas import tpu as pltpu
from jax.experimental.pallas import tpu_sc as plsc

VOCAB = 1000000
C0, C1 = 20000, 200000
D = 128
N0, D0 = 20000, 128
N1, D1 = 180000, 32
N2, D2 = 800000, 8

T = 4096 * 50
NW = 32
TPW = T // NW
L = 16

R0C = 128
R1C = 512
R2C = 1024

T0PAD = T + 256

BT = 8192


def _sc_gather(ids, emb0, emb1, emb2):
    mesh = plsc.VectorSubcoreMesh(core_axis_name="c", subcore_axis_name="s")

    @functools.partial(
        pl.kernel,
        out_type=(
            jax.ShapeDtypeStruct((T0PAD, D0), jnp.float32),
            jax.ShapeDtypeStruct((T0PAD, D1), jnp.float32),
            jax.ShapeDtypeStruct((T0PAD, D2), jnp.float32),
        ),
        mesh=mesh,
        compiler_params=pltpu.CompilerParams(
            use_tc_tiling_on_sc=False, needs_layout_passes=False),
        scratch_types=[
            pltpu.VMEM((TPW,), jnp.int32),
            pltpu.VMEM((TPW + R0C,), jnp.int32),
            pltpu.VMEM((TPW + R0C,), jnp.int32),
            pltpu.VMEM((TPW + R1C,), jnp.int32),
            pltpu.VMEM((TPW + R1C,), jnp.int32),
            pltpu.VMEM((TPW + R2C,), jnp.int32),
            pltpu.VMEM((TPW + R2C,), jnp.int32),
            pltpu.VMEM((R0C, D0), jnp.float32),
            pltpu.VMEM((R1C, D1), jnp.float32),
            pltpu.VMEM((R2C, D2), jnp.float32),
            pltpu.VMEM((R0C,), jnp.int32),
            pltpu.VMEM((R1C,), jnp.int32),
            pltpu.VMEM((R2C,), jnp.int32),
            pltpu.SemaphoreType.DMA,
        ],
    )
    def k(ids_hbm, e0_hbm, e1_hbm, e2_hbm, g0s_hbm, g1s_hbm, g2s_hbm,
          ids_v, p0_v, q0_v, p1_v, q1_v, p2_v, q2_v,
          rows0_v, rows1_v, rows2_v, pc0_v, pc1_v, pc2_v, sem):
        wid = lax.axis_index("s") * 2 + lax.axis_index("c")
        base = wid * TPW
        pltpu.sync_copy(ids_hbm.at[pl.ds(base, TPW)], ids_v)

        def idx_body(g, offs):
            o0, o1, o2 = offs
            v = ids_v[pl.ds(g * L, L)]
            pos = base + g * L + lax.iota(jnp.int32, L)
            m0 = (v < C0) & (v != 0)
            m1 = (v >= C0) & (v < C1)
            m2 = v >= C1
            outs = []
            for (m, p_v, q_v, o, sub) in (
                    (m0, p0_v, q0_v, o0, 0),
                    (m1, p1_v, q1_v, o1, C0),
                    (m2, p2_v, q2_v, o2, C1)):
                mi = m.astype(jnp.int32)
                tgt = o + plsc.cumsum(mi) - 1
                plsc.store_scatter(p_v, [tgt], pos, mask=m)
                plsc.store_scatter(q_v, [tgt], v - sub, mask=m)
                outs.append(o + jnp.sum(mi))
            return tuple(outs)

        o0, o1, o2 = lax.fori_loop(0, TPW // L, idx_body, (0, 0, 0))

        for (p_v, q_v, o, rc) in ((p0_v, q0_v, o0, R0C),
                                  (p1_v, q1_v, o1, R1C),
                                  (p2_v, q2_v, o2, R2C)):
            for u in range(rc // L):
                p_v[pl.ds(o + u * L, L)] = jnp.full((L,), T, jnp.int32)
                q_v[pl.ds(o + u * L, L)] = jnp.zeros((L,), jnp.int32)

        def make_round(e_hbm, gs_hbm, p_v, q_v, rows_v, pc_v, rc):
            def rnd(j, c):
                cb = j * rc
                pltpu.async_copy(
                    e_hbm.at[q_v.at[pl.ds(cb, rc)]], rows_v, sem).wait()
                for u in range(rc // L):
                    pc_v[pl.ds(u * L, L)] = p_v[pl.ds(cb + u * L, L)]
                pltpu.async_copy(rows_v, gs_hbm.at[pc_v], sem).wait()
                return c
            return rnd

        lax.fori_loop(0, (o0 + R0C - 1) // R0C,
                      make_round(e0_hbm, g0s_hbm, p0_v, q0_v, rows0_v,
                                 pc0_v, R0C), 0)
        lax.fori_loop(0, (o1 + R1C - 1) // R1C,
                      make_round(e1_hbm, g1s_hbm, p1_v, q1_v, rows1_v,
                                 pc1_v, R1C), 0)
        lax.fori_loop(0, (o2 + R2C - 1) // R2C,
                      make_round(e2_hbm, g2s_hbm, p2_v, q2_v, rows2_v,
                                 pc2_v, R2C), 0)

    return k(ids, emb0, emb1, emb2)


def _tc_combine_body(ids_ref, g0_ref, g1_ref, g2_ref, w1_ref, w2_ref,
                     b1_ref, b2_ref, o_ref):
    ids1 = ids_ref[...]
    idsb = jnp.broadcast_to(ids1, (ids1.shape[0], D))
    ids32 = idsb[:, :D1]
    ids8 = idsb[:, :D2]
    g0 = jnp.where((idsb < C0) & (idsb != 0), g0_ref[...], 0.0)
    g1 = jnp.where((ids32 >= C0) & (ids32 < C1) & (ids32 != C0),
                   g1_ref[...], 0.0)
    g2 = jnp.where(ids8 >= C1, g2_ref[...], 0.0)
    g2 = jnp.where(ids8 != C1, g2, 0.0)
    acc = g0
    acc = acc + jnp.dot(g1, w1_ref[...], preferred_element_type=jnp.float32)
    acc = acc + jnp.dot(g2, w2_ref[...], preferred_element_type=jnp.float32)
    acc = acc + jnp.where((idsb >= C0) & (idsb < C1), b1_ref[...], 0.0)
    acc = acc + jnp.where(idsb >= C1, b2_ref[...], 0.0)
    o_ref[...] = acc


def _tc_combine(ids2d, g0s, g1s, g2s, w1t, w2t, b1, b2):
    return pl.pallas_call(
        _tc_combine_body,
        grid=(T // BT,),
        in_specs=[
            pl.BlockSpec((BT, 1), lambda i: (i, 0)),
            pl.BlockSpec((BT, D0), lambda i: (i, 0)),
            pl.BlockSpec((BT, D1), lambda i: (i, 0)),
            pl.BlockSpec((BT, D2), lambda i: (i, 0)),
            pl.BlockSpec((D1, D), lambda i: (0, 0)),
            pl.BlockSpec((D2, D), lambda i: (0, 0)),
            pl.BlockSpec((1, D), lambda i: (0, 0)),
            pl.BlockSpec((1, D), lambda i: (0, 0)),
        ],
        out_specs=pl.BlockSpec((BT, D), lambda i: (i, 0)),
        out_shape=jax.ShapeDtypeStruct((T, D), jnp.float32),
    )(ids2d, g0s, g1s, g2s, w1t, w2t, b1, b2)


def kernel(input_ids, emb0, emb1, emb2, proj1_w, proj1_b, proj2_w, proj2_b):
    ids = input_ids.reshape(-1).astype(jnp.int32)
    g0s, g1s, g2s = _sc_gather(ids, emb0, emb1, emb2)
    out = _tc_combine(ids.reshape(T, 1), g0s, g1s, g2s,
                      proj1_w.T, proj2_w.T,
                      proj1_b.reshape(1, D), proj2_b.reshape(1, D))
    return out.reshape(input_ids.shape + (D,))
